# baseline (speedup 1.0000x reference)
import functools

import jax
import jax.numpy as jnp
from jax.experimental import pallas as pl
from jax.experimental.pallas import tpu as pltpu

NUM_EXPERTS = 16
TOPK = 2
BM = 1024


def _router_block(x_ref, w_ref, b_ref, logits_ref, wts_ref, idx_ref, mask_ref):
    xb = x_ref[...]
    w = w_ref[...]
    logits = jax.lax.dot_general(
        xb, w, (((1,), (1,)), ((), ())),
        preferred_element_type=jnp.float32)
    logits = logits + b_ref[...]
    logits_ref[...] = logits

    e_iota = jax.lax.broadcasted_iota(jnp.int32, logits.shape, 1)
    big = jnp.int32(NUM_EXPERTS)
    m1 = jnp.max(logits, axis=1, keepdims=True)
    i1 = jnp.min(jnp.where(logits == m1, e_iota, big), axis=1, keepdims=True)
    masked = jnp.where(e_iota == i1, -jnp.inf, logits)
    m2 = jnp.max(masked, axis=1, keepdims=True)
    i2 = jnp.min(jnp.where(masked == m2, e_iota, big), axis=1, keepdims=True)

    r = jnp.exp(m2 - m1)
    denom = 1.0 + r
    w1 = 1.0 / denom
    w2 = r / denom

    j2 = jax.lax.broadcasted_iota(jnp.int32, (xb.shape[0], TOPK), 1)
    wts_ref[...] = jnp.where(j2 == 0, w1, w2)
    idx_ref[...] = jnp.where(j2 == 0, i1, i2)

    e2 = jax.lax.broadcasted_iota(jnp.int32, (xb.shape[0], 2 * NUM_EXPERTS), 1)
    sel = jnp.where(e2 < NUM_EXPERTS, i1, i2)
    mask_ref[...] = (e2 % NUM_EXPERTS == sel).astype(jnp.int32)


@jax.jit
def kernel(x, W, b):
    M, K = x.shape
    E = W.shape[0]
    grid = (M // BM,)
    logits, wts, idx, mask = pl.pallas_call(
        _router_block,
        grid=grid,
        in_specs=[
            pl.BlockSpec((BM, K), lambda i: (i, 0)),
            pl.BlockSpec((E, K), lambda i: (0, 0)),
            pl.BlockSpec((1, E), lambda i: (0, 0)),
        ],
        out_specs=[
            pl.BlockSpec((BM, E), lambda i: (i, 0)),
            pl.BlockSpec((BM, TOPK), lambda i: (i, 0)),
            pl.BlockSpec((BM, TOPK), lambda i: (i, 0)),
            pl.BlockSpec((BM, TOPK * E), lambda i: (i, 0)),
        ],
        out_shape=[
            jax.ShapeDtypeStruct((M, E), jnp.float32),
            jax.ShapeDtypeStruct((M, TOPK), jnp.float32),
            jax.ShapeDtypeStruct((M, TOPK), jnp.int32),
            jax.ShapeDtypeStruct((M, TOPK * E), jnp.int32),
        ],
        compiler_params=pltpu.CompilerParams(
            dimension_semantics=("arbitrary",),
        ),
    )(x, W, b.reshape(1, E))
    return (logits, wts, idx, mask.reshape(M, TOPK, E))
